# baseline (speedup 1.0000x reference)
import functools

import jax
import jax.numpy as jnp
from jax import lax
from jax.experimental import pallas as pl
from jax.experimental.pallas import tpu as pltpu
from jax.experimental.pallas import tpu_sc as plsc

N = 10000
NPAD = 10240
F = 128
KIN_PAD = 256
NC, NS = 2, 16
NW = NC * NS
CH = 128
HALF = NPAD // 2
RPT = HALF // NS
ZCH = 64
NZ = RPT // ZCH
RBLK = 512
GRID = NPAD // RBLK

_mesh = plsc.VectorSubcoreMesh(
    core_axis_name="c", subcore_axis_name="s", num_cores=NC, num_subcores=NS)


DW = 16


def _deg_body(dst_hbm, out_hbm, idx_d, buf, acc16, sem0):
  kch = idx_d.shape[0]
  cid = lax.axis_index("c")
  sid = lax.axis_index("s")
  base = sid * RPT

  def fill(val):
    def body(r, _):
      buf[r, pl.ds(0, DW)] = jnp.full((DW,), val, jnp.float32)
      return _
    lax.fori_loop(0, CH, body, None)

  fill(0.0)
  for r in range(NZ):
    pltpu.sync_copy(buf.at[pl.ds(0, ZCH)],
                    acc16.at[pl.ds(base + r * ZCH, ZCH)])
  pltpu.sync_copy(dst_hbm.at[sid], idx_d)

  off = cid * HALF

  def remap_row(j, _):
    def remap_grp(g, __):
      d = idx_d[j, pl.ds(g * 16, 16)] - off
      bad = (d < 0) | (d >= HALF)
      idx_d[j, pl.ds(g * 16, 16)] = jnp.where(bad, HALF, d)
      return __
    lax.fori_loop(0, CH // 16, remap_grp, None)
    return _
  lax.fori_loop(0, kch, remap_row, None)
  fill(1.0)
  plsc.subcore_barrier()

  def step(j, _):
    pltpu.async_copy(buf, acc16.at[idx_d.at[j]], sem0, add=True)

    @pl.when(j >= 8)
    def _w():
      pltpu.make_async_copy(buf, acc16.at[idx_d.at[0]], sem0).wait()
    return _
  lax.fori_loop(0, kch, step, None)
  for _ in range(8):
    pltpu.make_async_copy(buf, acc16.at[idx_d.at[0]], sem0).wait()
  plsc.subcore_barrier()

  for r in range(NZ):
    pltpu.sync_copy(acc16.at[pl.ds(base + r * ZCH, ZCH)],
                    buf.at[pl.ds(0, ZCH)])
    pltpu.sync_copy(buf.at[pl.ds(0, ZCH)],
                    out_hbm.at[pl.ds(off + base + r * ZCH, ZCH)])


def _make_deg_kernel(kch):
  return pl.kernel(
      _deg_body,
      out_type=jax.ShapeDtypeStruct((NPAD, DW), jnp.float32),
      mesh=_mesh,
      scratch_types=[
          pltpu.VMEM((kch, CH), jnp.int32),
          pltpu.VMEM((CH, DW), jnp.float32),
          pltpu.VMEM_SHARED((HALF + CH, DW), jnp.float32),
          pltpu.SemaphoreType.DMA,
      ],
  )


BLK = 16


def _remap_block(idx_d4, p, off):
  def remap_row(r, _):
    def remap_grp(g, __):
      d = idx_d4[p, r, pl.ds(g * 16, 16)] - off
      bad = (d < 0) | (d >= HALF)
      idx_d4[p, r, pl.ds(g * 16, 16)] = jnp.where(bad, HALF, d)
      return __
    lax.fori_loop(0, CH // 16, remap_grp, None)
    return _
  lax.fori_loop(0, BLK, remap_row, None)


def _prop_body(ys_hbm, src_hbm, dst_hbm, zrow_hbm, out_hbm,
               idx_s2, idx_d4, rows4, acc_sh,
               gs0, gs1, gs2, gs3, ss0, ss1, ss2, ss3):
  tb = src_hbm.shape[1]
  gsem = (gs0, gs1, gs2, gs3)
  ssem = (ss0, ss1, ss2, ss3)
  cid = lax.axis_index("c")
  sid = lax.axis_index("s")
  base = sid * RPT
  off = cid * HALF

  pltpu.sync_copy(zrow_hbm, rows4.at[0])
  for r in range(NZ):
    pltpu.sync_copy(rows4.at[0, pl.ds(0, ZCH)],
                    acc_sh.at[pl.ds(base + r * ZCH, ZCH)])
  pltpu.sync_copy(src_hbm.at[sid, 0], idx_s2.at[0])
  pltpu.sync_copy(dst_hbm.at[sid, 0], idx_d4.at[0])
  _remap_block(idx_d4, 0, off)
  plsc.subcore_barrier()

  pltpu.async_copy(ys_hbm.at[idx_s2.at[0, 0]], rows4.at[0], gsem[0])
  pltpu.async_copy(ys_hbm.at[idx_s2.at[0, 1]], rows4.at[1], gsem[1])

  def block(ib, _):
    @pl.when(ib + 1 < tb)
    def _load():
      pltpu.sync_copy(src_hbm.at[sid, ib + 1], idx_s2.at[(ib + 1) % 2])
      pltpu.sync_copy(dst_hbm.at[sid, ib + 1], idx_d4.at[(ib + 1) % 4])
      _remap_block(idx_d4, (ib + 1) % 4, off)

    for b_ch in range(BLK):
      slot = b_ch % 4
      slot2 = (b_ch + 2) % 4

      def issue_next(b_ch=b_ch, slot2=slot2):
        pltpu.make_async_copy(
            rows4.at[slot2], acc_sh.at[idx_d4.at[0, 0]], ssem[slot2]).wait()
        if b_ch < BLK - 2:
          pltpu.async_copy(ys_hbm.at[idx_s2.at[ib % 2, b_ch + 2]],
                           rows4.at[slot2], gsem[slot2])
        else:
          pltpu.async_copy(ys_hbm.at[idx_s2.at[(ib + 1) % 2, b_ch + 2 - BLK]],
                           rows4.at[slot2], gsem[slot2])

      if b_ch < BLK - 2:
        if b_ch < 2:
          @pl.when(ib > 0)
          def _w():
            issue_next()

          @pl.when(ib == 0)
          def _nw():
            pltpu.async_copy(ys_hbm.at[idx_s2.at[0, b_ch + 2]],
                             rows4.at[slot2], gsem[slot2])
        else:
          issue_next()
      else:
        @pl.when(ib + 1 < tb)
        def _wn():
          issue_next()

      pltpu.make_async_copy(ys_hbm.at[idx_s2.at[0, 0]],
                            rows4.at[slot], gsem[slot]).wait()
      pltpu.async_copy(rows4.at[slot], acc_sh.at[idx_d4.at[ib % 4, b_ch]],
                       ssem[slot], add=True)
    return _

  lax.fori_loop(0, tb, block, None)

  for b in range(4):
    pltpu.make_async_copy(rows4.at[b], acc_sh.at[idx_d4.at[0, 0]],
                          ssem[b]).wait()
  plsc.subcore_barrier()

  for r in range(NZ):
    pltpu.sync_copy(acc_sh.at[pl.ds(base + r * ZCH, ZCH)],
                    rows4.at[0, pl.ds(0, ZCH)])
    pltpu.sync_copy(rows4.at[0, pl.ds(0, ZCH)],
                    out_hbm.at[pl.ds(off + base + r * ZCH, ZCH)])


def _make_prop_kernel(tb):
  return pl.kernel(
      _prop_body,
      out_type=jax.ShapeDtypeStruct((NPAD, F), jnp.float32),
      mesh=_mesh,
      scratch_types=[
          pltpu.VMEM((2, BLK, CH), jnp.int32),
          pltpu.VMEM((4, BLK, CH), jnp.int32),
          pltpu.VMEM((4, CH, F), jnp.float32),
          pltpu.VMEM_SHARED((HALF + CH, F), jnp.float32),
          pltpu.SemaphoreType.DMA,
          pltpu.SemaphoreType.DMA,
          pltpu.SemaphoreType.DMA,
          pltpu.SemaphoreType.DMA,
          pltpu.SemaphoreType.DMA,
          pltpu.SemaphoreType.DMA,
          pltpu.SemaphoreType.DMA,
          pltpu.SemaphoreType.DMA,
      ],
  )


def _mm0_body(x_ref, w_ref, degt_ref, ys_ref, dinv_ref):
  deg = degt_ref[...][:, 0:1] + 1.0
  dinv = lax.rsqrt(deg)
  xw = jnp.dot(x_ref[...], w_ref[...], preferred_element_type=jnp.float32)
  ys_ref[...] = xw * dinv
  dinv_ref[...] = dinv


def _mm0(x_pad, w0p, deg_t):
  return pl.pallas_call(
      _mm0_body,
      grid=(GRID,),
      in_specs=[
          pl.BlockSpec((RBLK, KIN_PAD), lambda i: (i, 0)),
          pl.BlockSpec((KIN_PAD, F), lambda i: (0, 0)),
          pl.BlockSpec((RBLK, DW), lambda i: (i, 0)),
      ],
      out_specs=[
          pl.BlockSpec((RBLK, F), lambda i: (i, 0)),
          pl.BlockSpec((RBLK, 1), lambda i: (i, 0)),
      ],
      out_shape=[
          jax.ShapeDtypeStruct((NPAD, F), jnp.float32),
          jax.ShapeDtypeStruct((NPAD, 1), jnp.float32),
      ],
  )(x_pad, w0p, deg_t)


def _layer_body(acc_ref, ys_ref, dinv_ref, b_ref, w_ref, out_ref):
  t = acc_ref[...] + ys_ref[...]
  dinv = dinv_ref[...]
  x = jnp.maximum(t * dinv + b_ref[...], 0.0)
  out_ref[...] = jnp.dot(
      x, w_ref[...], preferred_element_type=jnp.float32) * dinv


def _layer(acc, ys, dinv, b, w):
  return pl.pallas_call(
      _layer_body,
      grid=(GRID,),
      in_specs=[
          pl.BlockSpec((RBLK, F), lambda i: (i, 0)),
          pl.BlockSpec((RBLK, F), lambda i: (i, 0)),
          pl.BlockSpec((RBLK, 1), lambda i: (i, 0)),
          pl.BlockSpec((1, F), lambda i: (0, 0)),
          pl.BlockSpec((F, F), lambda i: (0, 0)),
      ],
      out_specs=pl.BlockSpec((RBLK, F), lambda i: (i, 0)),
      out_shape=jax.ShapeDtypeStruct((NPAD, F), jnp.float32),
  )(acc, ys, dinv, b, w)


def _final_body(acc_ref, ys_ref, dinv_ref, b_ref, out_ref):
  t = acc_ref[...] + ys_ref[...]
  out_ref[...] = jnp.maximum(t * dinv_ref[...] + b_ref[...], 0.0)


def _final(acc, ys, dinv, b):
  return pl.pallas_call(
      _final_body,
      grid=(GRID,),
      in_specs=[
          pl.BlockSpec((RBLK, F), lambda i: (i, 0)),
          pl.BlockSpec((RBLK, F), lambda i: (i, 0)),
          pl.BlockSpec((RBLK, 1), lambda i: (i, 0)),
          pl.BlockSpec((1, F), lambda i: (0, 0)),
      ],
      out_specs=pl.BlockSpec((RBLK, F), lambda i: (i, 0)),
      out_shape=jax.ShapeDtypeStruct((NPAD, F), jnp.float32),
  )(acc, ys, dinv, b)


@jax.jit
def kernel(h, edges, coords, W0, b0, W1, b1, W2, b2):
  e = edges.shape[1]
  tb = -(-e // (NS * CH * BLK))
  kch = tb * BLK
  epad = NS * kch * CH
  src_p = jnp.concatenate(
      [edges[0], jnp.full((epad - e,), N, jnp.int32)])
  dst_p = jnp.concatenate(
      [edges[1], jnp.full((epad - e,), N, jnp.int32)])
  src4 = src_p.reshape(NS, tb, BLK, CH)
  dst4 = dst_p.reshape(NS, tb, BLK, CH)
  dst3 = dst_p.reshape(NS, kch, CH)

  x_in = jnp.concatenate([h[0, 0], coords[0, 0]], axis=1)
  x_pad = jnp.pad(x_in, ((0, NPAD - N), (0, KIN_PAD - x_in.shape[1])))
  w0p = jnp.pad(W0, ((0, KIN_PAD - W0.shape[0]), (0, 0)))
  zrow = jnp.zeros((CH, F), jnp.float32)

  deg16 = _make_deg_kernel(kch)(dst3)

  ys0, dinv = _mm0(x_pad, w0p, deg16)
  prop = _make_prop_kernel(tb)

  acc = prop(ys0, src4, dst4, zrow)
  ys1 = _layer(acc, ys0, dinv, b0.reshape(1, F), W1)
  acc = prop(ys1, src4, dst4, zrow)
  ys2 = _layer(acc, ys1, dinv, b1.reshape(1, F), W2)
  acc = prop(ys2, src4, dst4, zrow)
  xf = _final(acc, ys2, dinv, b2.reshape(1, F))
  return xf[:N].reshape(1, 1, N, F)


# speedup vs baseline: 1.0026x; 1.0026x over previous
import functools

import jax
import jax.numpy as jnp
from jax import lax
from jax.experimental import pallas as pl
from jax.experimental.pallas import tpu as pltpu
from jax.experimental.pallas import tpu_sc as plsc

N = 10000
NPAD = 10240
F = 128
KIN_PAD = 256
NC, NS = 2, 16
NW = NC * NS
CH = 128
HALF = NPAD // 2
RPT = HALF // NS
ZCH = 64
NZ = RPT // ZCH
RBLK = 512
GRID = NPAD // RBLK

_mesh = plsc.VectorSubcoreMesh(
    core_axis_name="c", subcore_axis_name="s", num_cores=NC, num_subcores=NS)


DW = 16
OPB = 8
EPC = 2 * CH
DGB = 16


def _deg_body(dst_hbm, out_hbm, idx_d, buf, acc1, sem0):
  db = idx_d.shape[0]
  cid = lax.axis_index("c")
  sid = lax.axis_index("s")
  base = sid * RPT
  off = cid * HALF
  dlen = idx_d.shape[2]

  def fill(val):
    def body_r(r, _):
      buf[0, pl.ds(r * 16, 16)] = jnp.full((16,), val, jnp.float32)
      return _
    lax.fori_loop(0, dlen // 16, body_r, None)

  nblk = HALF // CH

  def zero_blk(b, _):
    @pl.when((b & (NS - 1)) == sid)
    def _z():
      o = pl.multiple_of(b * CH, CH)
      pltpu.sync_copy(buf.at[0, pl.ds(0, CH)], acc1.at[0, pl.ds(o, CH)])
    return _

  fill(0.0)
  lax.fori_loop(0, nblk, zero_blk, None)
  pltpu.sync_copy(dst_hbm.at[sid], idx_d)

  def remap_b(b, _):
    def remap_g(g, __):
      d = idx_d[b, 0, pl.ds(g * 16, 16)] - off
      bad = (d < 0) | (d >= HALF)
      idx_d[b, 0, pl.ds(g * 16, 16)] = jnp.where(bad, HALF, d)
      return __
    lax.fori_loop(0, dlen // 16, remap_g, None)
    return _
  lax.fori_loop(0, db, remap_b, None)
  fill(1.0)
  plsc.subcore_barrier()

  def step(b, _):
    pltpu.sync_copy(buf, acc1.at[idx_d.at[b]], add=True)
    return _
  lax.fori_loop(0, db, step, None)
  plsc.subcore_barrier()

  def out_blk(b, _):
    @pl.when((b & (NS - 1)) == sid)
    def _o():
      o = pl.multiple_of(b * CH, CH)
      pltpu.sync_copy(acc1.at[0, pl.ds(o, CH)], buf.at[0, pl.ds(0, CH)])
      pltpu.sync_copy(buf.at[0, pl.ds(0, CH)],
                      out_hbm.at[pl.ds(off + o, CH)])
    return _
  lax.fori_loop(0, nblk, out_blk, None)


def _make_deg_kernel(db):
  return pl.kernel(
      _deg_body,
      out_type=jax.ShapeDtypeStruct((NPAD,), jnp.float32),
      mesh=_mesh,
      scratch_types=[
          pltpu.VMEM((db, 1, DGB * CH), jnp.int32),
          pltpu.VMEM((1, DGB * CH), jnp.float32),
          pltpu.VMEM_SHARED((1, HALF + CH), jnp.float32),
          pltpu.SemaphoreType.DMA,
      ],
  )


def _prop_body(ys_hbm, src_hbm, dst_hbm, zrow_hbm, out_hbm,
               idx_s, idx_d, rows0, rows1, acc_sh, sem0, sem1):
  kch = idx_s.shape[0]
  cid = lax.axis_index("c")
  sid = lax.axis_index("s")
  base = sid * RPT
  off = cid * HALF

  pltpu.sync_copy(zrow_hbm, rows0)
  for r in range(NZ):
    pltpu.sync_copy(rows0.at[pl.ds(0, ZCH)],
                    acc_sh.at[pl.ds(base + r * ZCH, ZCH)])
  pltpu.sync_copy(src_hbm.at[sid], idx_s)
  pltpu.sync_copy(dst_hbm.at[sid], idx_d)

  def remap_row(j, _):
    def remap_grp(g, __):
      d = idx_d[j, pl.ds(g * 16, 16)] - off
      bad = (d < 0) | (d >= HALF)
      idx_d[j, pl.ds(g * 16, 16)] = jnp.where(bad, HALF, d)
      return __
    lax.fori_loop(0, CH // 16, remap_grp, None)
    return _
  lax.fori_loop(0, kch, remap_row, None)
  plsc.subcore_barrier()

  pltpu.async_copy(ys_hbm.at[idx_s.at[0]], rows0, sem0)

  def step(i, _):
    j0 = 2 * i
    j1 = j0 + 1
    pltpu.async_copy(ys_hbm.at[idx_s.at[j1]], rows1, sem1)
    pltpu.make_async_copy(ys_hbm.at[idx_s.at[j0]], rows0, sem0).wait()
    pltpu.sync_copy(rows0, acc_sh.at[idx_d.at[j0]], add=True)

    @pl.when(j0 + 2 < kch)
    def _():
      pltpu.async_copy(ys_hbm.at[idx_s.at[j0 + 2]], rows0, sem0)

    pltpu.make_async_copy(ys_hbm.at[idx_s.at[j1]], rows1, sem1).wait()
    pltpu.sync_copy(rows1, acc_sh.at[idx_d.at[j1]], add=True)
    return _

  lax.fori_loop(0, kch // 2, step, None)
  plsc.subcore_barrier()

  for r in range(NZ):
    pltpu.sync_copy(acc_sh.at[pl.ds(base + r * ZCH, ZCH)],
                    rows0.at[pl.ds(0, ZCH)])
    pltpu.sync_copy(rows0.at[pl.ds(0, ZCH)],
                    out_hbm.at[pl.ds(off + base + r * ZCH, ZCH)])


def _make_prop_kernel(kch):
  return pl.kernel(
      _prop_body,
      out_type=jax.ShapeDtypeStruct((NPAD, F), jnp.float32),
      mesh=_mesh,
      scratch_types=[
          pltpu.VMEM((kch, CH), jnp.int32),
          pltpu.VMEM((kch, CH), jnp.int32),
          pltpu.VMEM((CH, F), jnp.float32),
          pltpu.VMEM((CH, F), jnp.float32),
          pltpu.VMEM_SHARED((HALF + CH, F), jnp.float32),
          pltpu.SemaphoreType.DMA,
          pltpu.SemaphoreType.DMA,
      ],
  )


def _mm0_body(x_ref, w_ref, degt_ref, ys_ref, dinv_ref):
  deg = degt_ref[...] + 1.0
  dinv = lax.rsqrt(deg)
  xw = jnp.dot(x_ref[...], w_ref[...], preferred_element_type=jnp.float32)
  ys_ref[...] = xw * dinv
  dinv_ref[...] = dinv


def _mm0(x_pad, w0p, deg_t):
  return pl.pallas_call(
      _mm0_body,
      grid=(GRID,),
      in_specs=[
          pl.BlockSpec((RBLK, KIN_PAD), lambda i: (i, 0)),
          pl.BlockSpec((KIN_PAD, F), lambda i: (0, 0)),
          pl.BlockSpec((RBLK, 1), lambda i: (i, 0)),
      ],
      out_specs=[
          pl.BlockSpec((RBLK, F), lambda i: (i, 0)),
          pl.BlockSpec((RBLK, 1), lambda i: (i, 0)),
      ],
      out_shape=[
          jax.ShapeDtypeStruct((NPAD, F), jnp.float32),
          jax.ShapeDtypeStruct((NPAD, 1), jnp.float32),
      ],
  )(x_pad, w0p, deg_t)


def _layer_body(acc_ref, ys_ref, dinv_ref, b_ref, w_ref, out_ref):
  t = acc_ref[...] + ys_ref[...]
  dinv = dinv_ref[...]
  x = jnp.maximum(t * dinv + b_ref[...], 0.0)
  out_ref[...] = jnp.dot(
      x, w_ref[...], preferred_element_type=jnp.float32) * dinv


def _layer(acc, ys, dinv, b, w):
  return pl.pallas_call(
      _layer_body,
      grid=(GRID,),
      in_specs=[
          pl.BlockSpec((RBLK, F), lambda i: (i, 0)),
          pl.BlockSpec((RBLK, F), lambda i: (i, 0)),
          pl.BlockSpec((RBLK, 1), lambda i: (i, 0)),
          pl.BlockSpec((1, F), lambda i: (0, 0)),
          pl.BlockSpec((F, F), lambda i: (0, 0)),
      ],
      out_specs=pl.BlockSpec((RBLK, F), lambda i: (i, 0)),
      out_shape=jax.ShapeDtypeStruct((NPAD, F), jnp.float32),
  )(acc, ys, dinv, b, w)


def _final_body(acc_ref, ys_ref, dinv_ref, b_ref, out_ref):
  t = acc_ref[...] + ys_ref[...]
  out_ref[...] = jnp.maximum(t * dinv_ref[...] + b_ref[...], 0.0)


def _final(acc, ys, dinv, b):
  return pl.pallas_call(
      _final_body,
      grid=(GRID,),
      in_specs=[
          pl.BlockSpec((RBLK, F), lambda i: (i, 0)),
          pl.BlockSpec((RBLK, F), lambda i: (i, 0)),
          pl.BlockSpec((RBLK, 1), lambda i: (i, 0)),
          pl.BlockSpec((1, F), lambda i: (0, 0)),
      ],
      out_specs=pl.BlockSpec((RBLK, F), lambda i: (i, 0)),
      out_shape=jax.ShapeDtypeStruct((NPAD, F), jnp.float32),
  )(acc, ys, dinv, b)


@jax.jit
def kernel(h, edges, coords, W0, b0, W1, b1, W2, b2):
  e = edges.shape[1]
  nb = -(-e // (NS * OPB * 2 * CH))
  epad = NS * nb * OPB * 2 * CH
  db = epad // (NS * DGB * CH)
  src_p = jnp.concatenate(
      [edges[0], jnp.full((epad - e,), N, jnp.int32)])
  dst_p = jnp.concatenate(
      [edges[1], jnp.full((epad - e,), N, jnp.int32)])
  kch = epad // (NS * CH)
  src3 = src_p.reshape(NS, kch, CH)
  dst3 = dst_p.reshape(NS, kch, CH)
  dst_deg = dst_p.reshape(NS, db, 1, DGB * CH)

  x_in = jnp.concatenate([h[0, 0], coords[0, 0]], axis=1)
  x_pad = jnp.pad(x_in, ((0, NPAD - N), (0, KIN_PAD - x_in.shape[1])))
  w0p = jnp.pad(W0, ((0, KIN_PAD - W0.shape[0]), (0, 0)))
  zrow = jnp.zeros((CH, F), jnp.float32)

  deg1 = _make_deg_kernel(db)(dst_deg)
  deg_col = deg1.reshape(NPAD, 1)

  ys0, dinv = _mm0(x_pad, w0p, deg_col)
  prop = _make_prop_kernel(kch)

  acc = prop(ys0, src3, dst3, zrow)
  ys1 = _layer(acc, ys0, dinv, b0.reshape(1, F), W1)
  acc = prop(ys1, src3, dst3, zrow)
  ys2 = _layer(acc, ys1, dinv, b1.reshape(1, F), W2)
  acc = prop(ys2, src3, dst3, zrow)
  xf = _final(acc, ys2, dinv, b2.reshape(1, F))
  return xf[:N].reshape(1, 1, N, F)


# speedup vs baseline: 1.0036x; 1.0010x over previous
import functools

import jax
import jax.numpy as jnp
from jax import lax
from jax.experimental import pallas as pl
from jax.experimental.pallas import tpu as pltpu
from jax.experimental.pallas import tpu_sc as plsc

N = 10000
NPAD = 10240
F = 128
KIN_PAD = 256
NC, NS = 2, 16
NW = NC * NS
CH = 128
HALF = NPAD // 2
RPT = HALF // NS
ZCH = 64
NZ = RPT // ZCH
RBLK = 512
GRID = NPAD // RBLK

_mesh = plsc.VectorSubcoreMesh(
    core_axis_name="c", subcore_axis_name="s", num_cores=NC, num_subcores=NS)


DW = 16
OPB = 8
EPC = 2 * CH
DGB = 16


def _deg_body(dst_hbm, out_hbm, idx_d, buf, acc16, sem0):
  kch = idx_d.shape[0]
  cid = lax.axis_index("c")
  sid = lax.axis_index("s")
  base = sid * RPT
  off = cid * HALF

  def fill(val):
    def body_r(r, _):
      buf[r, pl.ds(0, DW)] = jnp.full((DW,), val, jnp.float32)
      return _
    lax.fori_loop(0, CH, body_r, None)

  fill(0.0)
  for r in range(NZ):
    pltpu.sync_copy(buf.at[pl.ds(0, ZCH)],
                    acc16.at[pl.ds(base + r * ZCH, ZCH)])
  pltpu.sync_copy(dst_hbm.at[sid], idx_d)

  def remap_b(b, _):
    def remap_g(g, __):
      d = idx_d[b, pl.ds(g * 16, 16)] - off
      bad = (d < 0) | (d >= HALF)
      idx_d[b, pl.ds(g * 16, 16)] = jnp.where(bad, HALF, d)
      return __
    lax.fori_loop(0, CH // 16, remap_g, None)
    return _
  lax.fori_loop(0, kch, remap_b, None)
  fill(1.0)
  plsc.subcore_barrier()

  def step(b, _):
    pltpu.async_copy(buf, acc16.at[idx_d.at[b]], sem0, add=True)

    @pl.when(b >= 4)
    def _w():
      pltpu.make_async_copy(buf, acc16.at[idx_d.at[0]], sem0).wait()
    return _
  lax.fori_loop(0, kch, step, None)
  for _ in range(4):
    pltpu.make_async_copy(buf, acc16.at[idx_d.at[0]], sem0).wait()
  plsc.subcore_barrier()

  for r in range(NZ):
    pltpu.sync_copy(acc16.at[pl.ds(base + r * ZCH, ZCH)],
                    buf.at[pl.ds(0, ZCH)])
    pltpu.sync_copy(buf.at[pl.ds(0, ZCH)],
                    out_hbm.at[pl.ds(off + base + r * ZCH, ZCH)])


def _make_deg_kernel(kch):
  return pl.kernel(
      _deg_body,
      out_type=jax.ShapeDtypeStruct((NPAD, DW), jnp.float32),
      mesh=_mesh,
      scratch_types=[
          pltpu.VMEM((kch, CH), jnp.int32),
          pltpu.VMEM((CH, DW), jnp.float32),
          pltpu.VMEM_SHARED((HALF + CH, DW), jnp.float32),
          pltpu.SemaphoreType.DMA,
      ],
  )


def _prop_body(ys_hbm, src_hbm, dst_hbm, zrow_hbm, out_hbm,
               idx_s, idx_d, rows0, rows1, acc_sh, sem0, sem1):
  kch = idx_s.shape[0]
  cid = lax.axis_index("c")
  sid = lax.axis_index("s")
  base = sid * RPT
  off = cid * HALF

  pltpu.sync_copy(zrow_hbm, rows0)
  for r in range(NZ):
    pltpu.sync_copy(rows0.at[pl.ds(0, ZCH)],
                    acc_sh.at[pl.ds(base + r * ZCH, ZCH)])
  pltpu.sync_copy(src_hbm.at[sid], idx_s)
  pltpu.sync_copy(dst_hbm.at[sid], idx_d)

  def remap_row(j, _):
    def remap_grp(g, __):
      d = idx_d[j, pl.ds(g * 16, 16)] - off
      bad = (d < 0) | (d >= HALF)
      idx_d[j, pl.ds(g * 16, 16)] = jnp.where(bad, HALF, d)
      return __
    lax.fori_loop(0, CH // 16, remap_grp, None)
    return _
  lax.fori_loop(0, kch, remap_row, None)
  plsc.subcore_barrier()

  pltpu.async_copy(ys_hbm.at[idx_s.at[0]], rows0, sem0)

  def step(i, _):
    j0 = 2 * i
    j1 = j0 + 1
    pltpu.async_copy(ys_hbm.at[idx_s.at[j1]], rows1, sem1)
    pltpu.make_async_copy(ys_hbm.at[idx_s.at[j0]], rows0, sem0).wait()
    pltpu.sync_copy(rows0, acc_sh.at[idx_d.at[j0]], add=True)

    @pl.when(j0 + 2 < kch)
    def _():
      pltpu.async_copy(ys_hbm.at[idx_s.at[j0 + 2]], rows0, sem0)

    pltpu.make_async_copy(ys_hbm.at[idx_s.at[j1]], rows1, sem1).wait()
    pltpu.sync_copy(rows1, acc_sh.at[idx_d.at[j1]], add=True)
    return _

  lax.fori_loop(0, kch // 2, step, None)
  plsc.subcore_barrier()

  for r in range(NZ):
    pltpu.sync_copy(acc_sh.at[pl.ds(base + r * ZCH, ZCH)],
                    rows0.at[pl.ds(0, ZCH)])
    pltpu.sync_copy(rows0.at[pl.ds(0, ZCH)],
                    out_hbm.at[pl.ds(off + base + r * ZCH, ZCH)])


def _make_prop_kernel(kch):
  return pl.kernel(
      _prop_body,
      out_type=jax.ShapeDtypeStruct((NPAD, F), jnp.float32),
      mesh=_mesh,
      scratch_types=[
          pltpu.VMEM((kch, CH), jnp.int32),
          pltpu.VMEM((kch, CH), jnp.int32),
          pltpu.VMEM((CH, F), jnp.float32),
          pltpu.VMEM((CH, F), jnp.float32),
          pltpu.VMEM_SHARED((HALF + CH, F), jnp.float32),
          pltpu.SemaphoreType.DMA,
          pltpu.SemaphoreType.DMA,
      ],
  )


def _mm0_body(x_ref, w_ref, degt_ref, ys_ref, dinv_ref):
  deg = degt_ref[...][:, 0:1] + 1.0
  dinv = lax.rsqrt(deg)
  xw = jnp.dot(x_ref[...], w_ref[...], preferred_element_type=jnp.float32)
  ys_ref[...] = xw * dinv
  dinv_ref[...] = dinv


def _mm0(x_pad, w0p, deg_t):
  return pl.pallas_call(
      _mm0_body,
      grid=(GRID,),
      in_specs=[
          pl.BlockSpec((RBLK, KIN_PAD), lambda i: (i, 0)),
          pl.BlockSpec((KIN_PAD, F), lambda i: (0, 0)),
          pl.BlockSpec((RBLK, DW), lambda i: (i, 0)),
      ],
      out_specs=[
          pl.BlockSpec((RBLK, F), lambda i: (i, 0)),
          pl.BlockSpec((RBLK, 1), lambda i: (i, 0)),
      ],
      out_shape=[
          jax.ShapeDtypeStruct((NPAD, F), jnp.float32),
          jax.ShapeDtypeStruct((NPAD, 1), jnp.float32),
      ],
  )(x_pad, w0p, deg_t)


def _layer_body(acc_ref, ys_ref, dinv_ref, b_ref, w_ref, out_ref):
  t = acc_ref[...] + ys_ref[...]
  dinv = dinv_ref[...]
  x = jnp.maximum(t * dinv + b_ref[...], 0.0)
  out_ref[...] = jnp.dot(
      x, w_ref[...], preferred_element_type=jnp.float32) * dinv


def _layer(acc, ys, dinv, b, w):
  return pl.pallas_call(
      _layer_body,
      grid=(GRID,),
      in_specs=[
          pl.BlockSpec((RBLK, F), lambda i: (i, 0)),
          pl.BlockSpec((RBLK, F), lambda i: (i, 0)),
          pl.BlockSpec((RBLK, 1), lambda i: (i, 0)),
          pl.BlockSpec((1, F), lambda i: (0, 0)),
          pl.BlockSpec((F, F), lambda i: (0, 0)),
      ],
      out_specs=pl.BlockSpec((RBLK, F), lambda i: (i, 0)),
      out_shape=jax.ShapeDtypeStruct((NPAD, F), jnp.float32),
  )(acc, ys, dinv, b, w)


def _final_body(acc_ref, ys_ref, dinv_ref, b_ref, out_ref):
  t = acc_ref[...] + ys_ref[...]
  out_ref[...] = jnp.maximum(t * dinv_ref[...] + b_ref[...], 0.0)


def _final(acc, ys, dinv, b):
  return pl.pallas_call(
      _final_body,
      grid=(GRID,),
      in_specs=[
          pl.BlockSpec((RBLK, F), lambda i: (i, 0)),
          pl.BlockSpec((RBLK, F), lambda i: (i, 0)),
          pl.BlockSpec((RBLK, 1), lambda i: (i, 0)),
          pl.BlockSpec((1, F), lambda i: (0, 0)),
      ],
      out_specs=pl.BlockSpec((RBLK, F), lambda i: (i, 0)),
      out_shape=jax.ShapeDtypeStruct((NPAD, F), jnp.float32),
  )(acc, ys, dinv, b)


@jax.jit
def kernel(h, edges, coords, W0, b0, W1, b1, W2, b2):
  e = edges.shape[1]
  nb = -(-e // (NS * OPB * 2 * CH))
  epad = NS * nb * OPB * 2 * CH
  db = epad // (NS * DGB * CH)
  src_p = jnp.concatenate(
      [edges[0], jnp.full((epad - e,), N, jnp.int32)])
  dst_p = jnp.concatenate(
      [edges[1], jnp.full((epad - e,), N, jnp.int32)])
  kch = epad // (NS * CH)
  src3 = src_p.reshape(NS, kch, CH)
  dst3 = dst_p.reshape(NS, kch, CH)
  dst_deg = dst_p.reshape(NS, kch, CH)

  x_in = jnp.concatenate([h[0, 0], coords[0, 0]], axis=1)
  x_pad = jnp.pad(x_in, ((0, NPAD - N), (0, KIN_PAD - x_in.shape[1])))
  w0p = jnp.pad(W0, ((0, KIN_PAD - W0.shape[0]), (0, 0)))
  zrow = jnp.zeros((CH, F), jnp.float32)

  deg16 = _make_deg_kernel(kch)(dst_deg)

  ys0, dinv = _mm0(x_pad, w0p, deg16)
  prop = _make_prop_kernel(kch)

  acc = prop(ys0, src3, dst3, zrow)
  ys1 = _layer(acc, ys0, dinv, b0.reshape(1, F), W1)
  acc = prop(ys1, src3, dst3, zrow)
  ys2 = _layer(acc, ys1, dinv, b1.reshape(1, F), W2)
  acc = prop(ys2, src3, dst3, zrow)
  xf = _final(acc, ys2, dinv, b2.reshape(1, F))
  return xf[:N].reshape(1, 1, N, F)


# speedup vs baseline: 1.4809x; 1.4757x over previous
import jax
import jax.numpy as jnp
from jax import lax
from jax.experimental import pallas as pl
from jax.experimental.pallas import tpu as pltpu
from jax.experimental.pallas import tpu_sc as plsc

N = 10000
NPAD = 10240
F = 128
KIN_PAD = 256
NC, NS = 2, 16
NW = NC * NS
CH = 128
HALF = NPAD // 2
RPT = HALF // NS
ZCH = 64
NZ = RPT // ZCH
RBLK = 512
GRID = NPAD // RBLK

_mesh = plsc.VectorSubcoreMesh(
    core_axis_name="c", subcore_axis_name="s", num_cores=NC, num_subcores=NS)


DW = 16


def _deg_body(dst_hbm, out_hbm, idx_d, buf, acc16, sem0):
  kch = idx_d.shape[0]
  cid = lax.axis_index("c")
  sid = lax.axis_index("s")
  base = sid * RPT
  off = cid * HALF

  def fill(val):
    def body_r(r, _):
      buf[r, pl.ds(0, DW)] = jnp.full((DW,), val, jnp.float32)
      return _
    lax.fori_loop(0, CH, body_r, None)

  fill(0.0)
  for r in range(NZ):
    pltpu.sync_copy(buf.at[pl.ds(0, ZCH)],
                    acc16.at[pl.ds(base + r * ZCH, ZCH)])
  pltpu.sync_copy(dst_hbm.at[sid], idx_d)

  def remap_b(b, _):
    def remap_g(g, __):
      d = idx_d[b, pl.ds(g * 16, 16)] - off
      bad = (d < 0) | (d >= HALF)
      idx_d[b, pl.ds(g * 16, 16)] = jnp.where(bad, HALF, d)
      return __
    lax.fori_loop(0, CH // 16, remap_g, None)
    return _
  lax.fori_loop(0, kch, remap_b, None)
  fill(1.0)
  plsc.subcore_barrier()

  def step(b, _):
    pltpu.sync_copy(buf, acc16.at[idx_d.at[b]], add=True)
    return _
  lax.fori_loop(0, kch, step, None)
  plsc.subcore_barrier()

  for r in range(NZ):
    pltpu.sync_copy(acc16.at[pl.ds(base + r * ZCH, ZCH)],
                    buf.at[pl.ds(0, ZCH)])
    pltpu.sync_copy(buf.at[pl.ds(0, ZCH)],
                    out_hbm.at[pl.ds(off + base + r * ZCH, ZCH)])


def _make_deg_kernel(kch):
  return pl.kernel(
      _deg_body,
      out_type=jax.ShapeDtypeStruct((NPAD, DW), jnp.float32),
      mesh=_mesh,
      scratch_types=[
          pltpu.VMEM((kch, CH), jnp.int32),
          pltpu.VMEM((CH, DW), jnp.float32),
          pltpu.VMEM_SHARED((HALF + CH, DW), jnp.float32),
          pltpu.SemaphoreType.DMA,
      ],
  )


def _prop_body(ys_hbm, src_hbm, dst_hbm, zrow_hbm, out_hbm,
               idx_s, idx_d, rows0, rows1, acc_sh, sem0, sem1):
  kch = idx_s.shape[0]
  cid = lax.axis_index("c")
  sid = lax.axis_index("s")
  base = sid * RPT
  off = cid * HALF

  pltpu.sync_copy(zrow_hbm, rows0)
  for r in range(NZ):
    pltpu.sync_copy(rows0.at[pl.ds(0, ZCH)],
                    acc_sh.at[pl.ds(base + r * ZCH, ZCH)])
  pltpu.sync_copy(src_hbm.at[sid], idx_s)
  pltpu.sync_copy(dst_hbm.at[sid], idx_d)

  def remap_row(j, _):
    def remap_grp(g, __):
      d = idx_d[j, pl.ds(g * 16, 16)] - off
      bad = (d < 0) | (d >= HALF)
      idx_d[j, pl.ds(g * 16, 16)] = jnp.where(bad, HALF, d)
      return __
    lax.fori_loop(0, CH // 16, remap_grp, None)
    return _
  lax.fori_loop(0, kch, remap_row, None)
  plsc.subcore_barrier()

  pltpu.async_copy(ys_hbm.at[idx_s.at[0]], rows0, sem0)

  def step(i, _):
    j0 = 2 * i
    j1 = j0 + 1
    pltpu.async_copy(ys_hbm.at[idx_s.at[j1]], rows1, sem1)
    pltpu.make_async_copy(ys_hbm.at[idx_s.at[j0]], rows0, sem0).wait()
    pltpu.sync_copy(rows0, acc_sh.at[idx_d.at[j0]], add=True)

    @pl.when(j0 + 2 < kch)
    def _():
      pltpu.async_copy(ys_hbm.at[idx_s.at[j0 + 2]], rows0, sem0)

    pltpu.make_async_copy(ys_hbm.at[idx_s.at[j1]], rows1, sem1).wait()
    pltpu.sync_copy(rows1, acc_sh.at[idx_d.at[j1]], add=True)
    return _

  lax.fori_loop(0, kch // 2, step, None)
  plsc.subcore_barrier()

  for r in range(NZ):
    pltpu.sync_copy(acc_sh.at[pl.ds(base + r * ZCH, ZCH)],
                    rows0.at[pl.ds(0, ZCH)])
    pltpu.sync_copy(rows0.at[pl.ds(0, ZCH)],
                    out_hbm.at[pl.ds(off + base + r * ZCH, ZCH)])


def _make_prop_kernel(kch):
  return pl.kernel(
      _prop_body,
      out_type=jax.ShapeDtypeStruct((NPAD, F), jnp.float32),
      mesh=_mesh,
      scratch_types=[
          pltpu.VMEM((kch, CH), jnp.int32),
          pltpu.VMEM((kch, CH), jnp.int32),
          pltpu.VMEM((CH, F), jnp.float32),
          pltpu.VMEM((CH, F), jnp.float32),
          pltpu.VMEM_SHARED((HALF + CH, F), jnp.float32),
          pltpu.SemaphoreType.DMA,
          pltpu.SemaphoreType.DMA,
      ],
  )


def _mm0_body(x_ref, w_ref, degt_ref, ys_ref, dinv_ref):
  deg = degt_ref[...][:, 0:1] + 1.0
  dinv = lax.rsqrt(deg)
  xw = jnp.dot(x_ref[...], w_ref[...], preferred_element_type=jnp.float32)
  ys_ref[...] = xw * dinv
  dinv_ref[...] = dinv


def _mm0(x_pad, w0p, deg_t):
  return pl.pallas_call(
      _mm0_body,
      grid=(GRID,),
      in_specs=[
          pl.BlockSpec((RBLK, KIN_PAD), lambda i: (i, 0)),
          pl.BlockSpec((KIN_PAD, F), lambda i: (0, 0)),
          pl.BlockSpec((RBLK, DW), lambda i: (i, 0)),
      ],
      out_specs=[
          pl.BlockSpec((RBLK, F), lambda i: (i, 0)),
          pl.BlockSpec((RBLK, 1), lambda i: (i, 0)),
      ],
      out_shape=[
          jax.ShapeDtypeStruct((NPAD, F), jnp.float32),
          jax.ShapeDtypeStruct((NPAD, 1), jnp.float32),
      ],
  )(x_pad, w0p, deg_t)


def _layer_body(acc_ref, ys_ref, dinv_ref, b_ref, w_ref, out_ref):
  t = acc_ref[...] + ys_ref[...]
  dinv = dinv_ref[...]
  x = jnp.maximum(t * dinv + b_ref[...], 0.0)
  out_ref[...] = jnp.dot(
      x, w_ref[...], preferred_element_type=jnp.float32) * dinv


def _layer(acc, ys, dinv, b, w):
  return pl.pallas_call(
      _layer_body,
      grid=(GRID,),
      in_specs=[
          pl.BlockSpec((RBLK, F), lambda i: (i, 0)),
          pl.BlockSpec((RBLK, F), lambda i: (i, 0)),
          pl.BlockSpec((RBLK, 1), lambda i: (i, 0)),
          pl.BlockSpec((1, F), lambda i: (0, 0)),
          pl.BlockSpec((F, F), lambda i: (0, 0)),
      ],
      out_specs=pl.BlockSpec((RBLK, F), lambda i: (i, 0)),
      out_shape=jax.ShapeDtypeStruct((NPAD, F), jnp.float32),
  )(acc, ys, dinv, b, w)


def _final_body(acc_ref, ys_ref, dinv_ref, b_ref, out_ref):
  t = acc_ref[...] + ys_ref[...]
  out_ref[...] = jnp.maximum(t * dinv_ref[...] + b_ref[...], 0.0)


def _final(acc, ys, dinv, b):
  return pl.pallas_call(
      _final_body,
      grid=(GRID,),
      in_specs=[
          pl.BlockSpec((RBLK, F), lambda i: (i, 0)),
          pl.BlockSpec((RBLK, F), lambda i: (i, 0)),
          pl.BlockSpec((RBLK, 1), lambda i: (i, 0)),
          pl.BlockSpec((1, F), lambda i: (0, 0)),
      ],
      out_specs=pl.BlockSpec((RBLK, F), lambda i: (i, 0)),
      out_shape=jax.ShapeDtypeStruct((NPAD, F), jnp.float32),
  )(acc, ys, dinv, b)


@jax.jit
def kernel(h, edges, coords, W0, b0, W1, b1, W2, b2):
  e = edges.shape[1]
  kch = 2 * -(-e // (2 * NS * CH))
  epad = NS * kch * CH
  src_p = jnp.concatenate(
      [edges[0], jnp.full((epad - e,), N, jnp.int32)])
  dst_p = jnp.concatenate(
      [edges[1], jnp.full((epad - e,), N, jnp.int32)])
  src3 = src_p.reshape(NS, kch, CH)
  dst3 = dst_p.reshape(NS, kch, CH)
  dst_deg = dst_p.reshape(NS, kch, CH)

  x_in = jnp.concatenate([h[0, 0], coords[0, 0]], axis=1)
  x_pad = jnp.pad(x_in, ((0, NPAD - N), (0, KIN_PAD - x_in.shape[1])))
  w0p = jnp.pad(W0, ((0, KIN_PAD - W0.shape[0]), (0, 0)))
  zrow = jnp.zeros((CH, F), jnp.float32)

  deg16 = _make_deg_kernel(kch)(dst_deg)

  ys0, dinv = _mm0(x_pad, w0p, deg16)
  prop = _make_prop_kernel(kch)

  acc = prop(ys0, src3, dst3, zrow)
  ys1 = _layer(acc, ys0, dinv, b0.reshape(1, F), W1)
  acc = prop(ys1, src3, dst3, zrow)
  ys2 = _layer(acc, ys1, dinv, b1.reshape(1, F), W2)
  acc = prop(ys2, src3, dst3, zrow)
  xf = _final(acc, ys2, dinv, b2.reshape(1, F))
  return xf[:N].reshape(1, 1, N, F)


# speedup vs baseline: 1.7733x; 1.1974x over previous
import jax
import jax.numpy as jnp
from jax import lax
from jax.experimental import pallas as pl
from jax.experimental.pallas import tpu as pltpu
from jax.experimental.pallas import tpu_sc as plsc

N = 10000
NPAD = 10240
F = 128
KIN_PAD = 256
NC, NS = 2, 16
NW = NC * NS
CH = 128
HALF = NPAD // 2
RPT = HALF // NS
ZCH = 64
NZ = RPT // ZCH
RBLK = 512
GRID = NPAD // RBLK

_mesh = plsc.VectorSubcoreMesh(
    core_axis_name="c", subcore_axis_name="s", num_cores=NC, num_subcores=NS)


DW = 16


def _deg_body(dst_hbm, out_hbm, idx_d, buf, acc16, sem0):
  kch = idx_d.shape[0]
  cid = lax.axis_index("c")
  sid = lax.axis_index("s")
  base = sid * RPT
  off = cid * HALF

  def fill(val):
    def body_r(r, _):
      buf[r, pl.ds(0, DW)] = jnp.full((DW,), val, jnp.float32)
      return _
    lax.fori_loop(0, CH, body_r, None)

  fill(0.0)
  for r in range(NZ):
    pltpu.sync_copy(buf.at[pl.ds(0, ZCH)],
                    acc16.at[pl.ds(base + r * ZCH, ZCH)])
  pltpu.sync_copy(dst_hbm.at[sid], idx_d)

  iota16 = lax.iota(jnp.int32, 16)

  def remap_b(b, _):
    def remap_g(g, __):
      d = idx_d[b, pl.ds(g * 16, 16)] - off
      bad = (d < 0) | (d >= HALF)
      trash = HALF + (g & 7) * 16 + iota16
      idx_d[b, pl.ds(g * 16, 16)] = jnp.where(bad, trash, d)
      return __
    lax.fori_loop(0, CH // 16, remap_g, None)
    return _
  lax.fori_loop(0, kch, remap_b, None)
  fill(1.0)
  plsc.subcore_barrier()

  def step(b, _):
    pltpu.sync_copy(buf, acc16.at[idx_d.at[b]], add=True)
    return _
  lax.fori_loop(0, kch, step, None)
  plsc.subcore_barrier()

  for r in range(NZ):
    pltpu.sync_copy(acc16.at[pl.ds(base + r * ZCH, ZCH)],
                    buf.at[pl.ds(0, ZCH)])
    pltpu.sync_copy(buf.at[pl.ds(0, ZCH)],
                    out_hbm.at[pl.ds(off + base + r * ZCH, ZCH)])


def _make_deg_kernel(kch):
  return pl.kernel(
      _deg_body,
      out_type=jax.ShapeDtypeStruct((NPAD, DW), jnp.float32),
      mesh=_mesh,
      scratch_types=[
          pltpu.VMEM((kch, CH), jnp.int32),
          pltpu.VMEM((CH, DW), jnp.float32),
          pltpu.VMEM_SHARED((HALF + CH, DW), jnp.float32),
          pltpu.SemaphoreType.DMA,
      ],
  )


def _prop_body(ys_hbm, src_hbm, dst_hbm, zrow_hbm, out_hbm,
               idx_s, idx_d, rows0, rows1, acc_sh, sem0, sem1):
  kch = idx_s.shape[0]
  cid = lax.axis_index("c")
  sid = lax.axis_index("s")
  base = sid * RPT
  off = cid * HALF

  pltpu.sync_copy(zrow_hbm, rows0)
  for r in range(NZ):
    pltpu.sync_copy(rows0.at[pl.ds(0, ZCH)],
                    acc_sh.at[pl.ds(base + r * ZCH, ZCH)])
  pltpu.sync_copy(src_hbm.at[sid], idx_s)
  pltpu.sync_copy(dst_hbm.at[sid], idx_d)

  iota16 = lax.iota(jnp.int32, 16)

  def remap_row(j, _):
    def remap_grp(g, __):
      d = idx_d[j, pl.ds(g * 16, 16)] - off
      bad = (d < 0) | (d >= HALF)
      trash = HALF + (g & 7) * 16 + iota16
      idx_d[j, pl.ds(g * 16, 16)] = jnp.where(bad, trash, d)
      return __
    lax.fori_loop(0, CH // 16, remap_grp, None)
    return _
  lax.fori_loop(0, kch, remap_row, None)
  plsc.subcore_barrier()

  pltpu.async_copy(ys_hbm.at[idx_s.at[0]], rows0, sem0)

  def step(i, _):
    j0 = 2 * i
    j1 = j0 + 1
    pltpu.async_copy(ys_hbm.at[idx_s.at[j1]], rows1, sem1)
    pltpu.make_async_copy(ys_hbm.at[idx_s.at[j0]], rows0, sem0).wait()
    pltpu.sync_copy(rows0, acc_sh.at[idx_d.at[j0]], add=True)

    @pl.when(j0 + 2 < kch)
    def _():
      pltpu.async_copy(ys_hbm.at[idx_s.at[j0 + 2]], rows0, sem0)

    pltpu.make_async_copy(ys_hbm.at[idx_s.at[j1]], rows1, sem1).wait()
    pltpu.sync_copy(rows1, acc_sh.at[idx_d.at[j1]], add=True)
    return _

  lax.fori_loop(0, kch // 2, step, None)
  plsc.subcore_barrier()

  for r in range(NZ):
    pltpu.sync_copy(acc_sh.at[pl.ds(base + r * ZCH, ZCH)],
                    rows0.at[pl.ds(0, ZCH)])
    pltpu.sync_copy(rows0.at[pl.ds(0, ZCH)],
                    out_hbm.at[pl.ds(off + base + r * ZCH, ZCH)])


def _make_prop_kernel(kch):
  return pl.kernel(
      _prop_body,
      out_type=jax.ShapeDtypeStruct((NPAD, F), jnp.float32),
      mesh=_mesh,
      scratch_types=[
          pltpu.VMEM((kch, CH), jnp.int32),
          pltpu.VMEM((kch, CH), jnp.int32),
          pltpu.VMEM((CH, F), jnp.float32),
          pltpu.VMEM((CH, F), jnp.float32),
          pltpu.VMEM_SHARED((HALF + CH, F), jnp.float32),
          pltpu.SemaphoreType.DMA,
          pltpu.SemaphoreType.DMA,
      ],
  )


def _mm0_body(x_ref, w_ref, degt_ref, ys_ref, dinv_ref):
  deg = degt_ref[...][:, 0:1] + 1.0
  dinv = lax.rsqrt(deg)
  xw = jnp.dot(x_ref[...], w_ref[...], preferred_element_type=jnp.float32)
  ys_ref[...] = xw * dinv
  dinv_ref[...] = dinv


def _mm0(x_pad, w0p, deg_t):
  return pl.pallas_call(
      _mm0_body,
      grid=(GRID,),
      in_specs=[
          pl.BlockSpec((RBLK, KIN_PAD), lambda i: (i, 0)),
          pl.BlockSpec((KIN_PAD, F), lambda i: (0, 0)),
          pl.BlockSpec((RBLK, DW), lambda i: (i, 0)),
      ],
      out_specs=[
          pl.BlockSpec((RBLK, F), lambda i: (i, 0)),
          pl.BlockSpec((RBLK, 1), lambda i: (i, 0)),
      ],
      out_shape=[
          jax.ShapeDtypeStruct((NPAD, F), jnp.float32),
          jax.ShapeDtypeStruct((NPAD, 1), jnp.float32),
      ],
  )(x_pad, w0p, deg_t)


def _layer_body(acc_ref, ys_ref, dinv_ref, b_ref, w_ref, out_ref):
  t = acc_ref[...] + ys_ref[...]
  dinv = dinv_ref[...]
  x = jnp.maximum(t * dinv + b_ref[...], 0.0)
  out_ref[...] = jnp.dot(
      x, w_ref[...], preferred_element_type=jnp.float32) * dinv


def _layer(acc, ys, dinv, b, w):
  return pl.pallas_call(
      _layer_body,
      grid=(GRID,),
      in_specs=[
          pl.BlockSpec((RBLK, F), lambda i: (i, 0)),
          pl.BlockSpec((RBLK, F), lambda i: (i, 0)),
          pl.BlockSpec((RBLK, 1), lambda i: (i, 0)),
          pl.BlockSpec((1, F), lambda i: (0, 0)),
          pl.BlockSpec((F, F), lambda i: (0, 0)),
      ],
      out_specs=pl.BlockSpec((RBLK, F), lambda i: (i, 0)),
      out_shape=jax.ShapeDtypeStruct((NPAD, F), jnp.float32),
  )(acc, ys, dinv, b, w)


def _final_body(acc_ref, ys_ref, dinv_ref, b_ref, out_ref):
  t = acc_ref[...] + ys_ref[...]
  out_ref[...] = jnp.maximum(t * dinv_ref[...] + b_ref[...], 0.0)


def _final(acc, ys, dinv, b):
  return pl.pallas_call(
      _final_body,
      grid=(GRID,),
      in_specs=[
          pl.BlockSpec((RBLK, F), lambda i: (i, 0)),
          pl.BlockSpec((RBLK, F), lambda i: (i, 0)),
          pl.BlockSpec((RBLK, 1), lambda i: (i, 0)),
          pl.BlockSpec((1, F), lambda i: (0, 0)),
      ],
      out_specs=pl.BlockSpec((RBLK, F), lambda i: (i, 0)),
      out_shape=jax.ShapeDtypeStruct((NPAD, F), jnp.float32),
  )(acc, ys, dinv, b)


@jax.jit
def kernel(h, edges, coords, W0, b0, W1, b1, W2, b2):
  e = edges.shape[1]
  kch = 2 * -(-e // (2 * NS * CH))
  epad = NS * kch * CH
  src_p = jnp.concatenate(
      [edges[0], jnp.full((epad - e,), N, jnp.int32)])
  dst_p = jnp.concatenate(
      [edges[1], jnp.full((epad - e,), N, jnp.int32)])
  src3 = src_p.reshape(NS, kch, CH)
  dst3 = dst_p.reshape(NS, kch, CH)
  dst_deg = dst_p.reshape(NS, kch, CH)

  x_in = jnp.concatenate([h[0, 0], coords[0, 0]], axis=1)
  x_pad = jnp.pad(x_in, ((0, NPAD - N), (0, KIN_PAD - x_in.shape[1])))
  w0p = jnp.pad(W0, ((0, KIN_PAD - W0.shape[0]), (0, 0)))
  zrow = jnp.zeros((CH, F), jnp.float32)

  deg16 = _make_deg_kernel(kch)(dst_deg)

  ys0, dinv = _mm0(x_pad, w0p, deg16)
  prop = _make_prop_kernel(kch)

  acc = prop(ys0, src3, dst3, zrow)
  ys1 = _layer(acc, ys0, dinv, b0.reshape(1, F), W1)
  acc = prop(ys1, src3, dst3, zrow)
  ys2 = _layer(acc, ys1, dinv, b1.reshape(1, F), W2)
  acc = prop(ys2, src3, dst3, zrow)
  xf = _final(acc, ys2, dinv, b2.reshape(1, F))
  return xf[:N].reshape(1, 1, N, F)
